# baseline (speedup 1.0000x reference)
import functools
import math
import jax
import jax.numpy as jnp
import numpy as np
from jax import lax
from jax.experimental import pallas as pl
from jax.experimental.pallas import tpu as pltpu
from jax.experimental.pallas import tpu_sc as plsc

DIM = 40
HALF = DIM // 2
BLOCK = 1024
OUT_W = 6 * DIM

NC, NS = 2, 16
NW = NC * NS


def _sin_turns(u):
    u = u - jnp.floor(u)
    t = 4.0 * u
    q = jnp.floor(t + 0.5)
    theta = (t - q) * (math.pi / 2.0)
    th2 = theta * theta
    s = -1.0 / 5040.0
    s = s * th2 + 1.0 / 120.0
    s = s * th2 - 1.0 / 6.0
    s = s * th2 + 1.0
    s = s * theta
    c = -1.0 / 720.0
    c = c * th2 + 1.0 / 24.0
    c = c * th2 - 1.0 / 2.0
    c = c * th2 + 1.0
    qm = q.astype(jnp.int32) & 3
    mag = jnp.where((qm & 1) == 1, c, s)
    return jnp.where(qm >= 2, -mag, mag)


def _dense_block(xyz_ref, cat_ref, fm_ref, off_ref, out_ref):
    u = jnp.dot(xyz_ref[...], fm_ref[...],
                preferred_element_type=jnp.float32,
                precision=jax.lax.Precision.HIGHEST) + off_ref[...]
    out_ref[:, 0:DIM] = jnp.zeros((xyz_ref.shape[0], DIM), jnp.float32)
    out_ref[:, DIM:4 * DIM] = _sin_turns(u)
    out_ref[:, 4 * DIM:5 * DIM] = cat_ref[...]
    out_ref[:, 5 * DIM:6 * DIM] = jnp.zeros((xyz_ref.shape[0], DIM),
                                            jnp.float32)


def _make_sc_gather(n):
    rows_w = n // NW
    mesh = plsc.VectorSubcoreMesh(core_axis_name="c", subcore_axis_name="s",
                                  num_cores=NC, num_subcores=NS)

    @functools.partial(
        pl.kernel, mesh=mesh,
        compiler_params=pltpu.CompilerParams(use_tc_tiling_on_sc=False),
        out_type=(),
        scratch_types=[
            pltpu.VMEM((rows_w,), jnp.int32),
            pltpu.VMEM((rows_w,), jnp.int32),
            pltpu.VMEM((rows_w, DIM), jnp.float32),
            pltpu.VMEM((rows_w, DIM), jnp.float32),
            pltpu.SemaphoreType.DMA,
        ],
    )
    def sc_gather(names_hbm, num_hbm, at_hbm, nt_hbm, out_hbm,
                  idx_a, idx_n, rows_a, rows_n, sem):
        wid = lax.axis_index("s") * NC + lax.axis_index("c")
        base = wid * rows_w
        pltpu.sync_copy(names_hbm.at[pl.ds(base, rows_w)], idx_a)
        pltpu.sync_copy(num_hbm.at[pl.ds(base, rows_w)], idx_n)
        ca = pltpu.async_copy(at_hbm.at[idx_a], rows_a, sem)
        cb = pltpu.async_copy(nt_hbm.at[idx_n], rows_n, sem)
        ca.wait()
        cb.wait()
        pltpu.sync_copy(rows_a, out_hbm.at[pl.ds(base, rows_w),
                                           pl.ds(0, DIM)])
        pltpu.sync_copy(rows_n, out_hbm.at[pl.ds(base, rows_w),
                                           pl.ds(5 * DIM, DIM)])

    return sc_gather


def kernel(names, x, y, z, categorical, numerical, atom_table, num_table):
    n = names.shape[0]

    inv = (10000.0 ** (-2.0 * np.arange(HALF) / DIM)).astype(np.float32)
    fm = np.zeros((3, 3 * DIM), dtype=np.float32)
    off = np.zeros((1, 3 * DIM), dtype=np.float32)
    for j in range(3):
        fm[j, j * DIM:j * DIM + HALF] = inv
        fm[j, j * DIM + HALF:(j + 1) * DIM] = inv
        off[0, j * DIM + HALF:(j + 1) * DIM] = 0.25
    fm = jnp.asarray(fm)
    off = jnp.asarray(off)
    xyz = jnp.concatenate([x, y, z], axis=1)

    row_spec = lambda w: pl.BlockSpec((BLOCK, w), lambda i: (i, 0))
    fix_spec = lambda h, w: pl.BlockSpec((h, w), lambda i: (0, 0))

    dense = pl.pallas_call(
        _dense_block,
        grid=(n // BLOCK,),
        in_specs=[
            row_spec(3),
            row_spec(DIM),
            fix_spec(3, 3 * DIM),
            fix_spec(1, 3 * DIM),
        ],
        out_specs=row_spec(OUT_W),
        out_shape=jax.ShapeDtypeStruct((n, OUT_W), jnp.float32),
        compiler_params=pltpu.CompilerParams(
            dimension_semantics=("arbitrary",)),
    )(xyz, categorical, fm, off)

    out_ref = jax.new_ref(dense)
    _make_sc_gather(n)(names, numerical, atom_table, num_table, out_ref)
    return jax.freeze(out_ref)


# speedup vs baseline: 1.4068x; 1.4068x over previous
import functools
import math
import jax
import jax.numpy as jnp
import numpy as np
from jax import lax
from jax.experimental import pallas as pl
from jax.experimental.pallas import tpu as pltpu
from jax.experimental.pallas import tpu_sc as plsc

DIM = 40
HALF = DIM // 2
BLOCK = 1024

NC, NS = 2, 16
NW = NC * NS


def _sin_turns(u):
    u = u - jnp.floor(u)
    t = 4.0 * u
    q = jnp.floor(t + 0.5)
    theta = (t - q) * (math.pi / 2.0)
    th2 = theta * theta
    s = -1.0 / 5040.0
    s = s * th2 + 1.0 / 120.0
    s = s * th2 - 1.0 / 6.0
    s = s * th2 + 1.0
    s = s * theta
    c = -1.0 / 720.0
    c = c * th2 + 1.0 / 24.0
    c = c * th2 - 1.0 / 2.0
    c = c * th2 + 1.0
    qm = q.astype(jnp.int32) & 3
    mag = jnp.where((qm & 1) == 1, c, s)
    return jnp.where(qm >= 2, -mag, mag)


def _make_sc_gather(n2):
    rows_w = n2 // NW
    mesh = plsc.VectorSubcoreMesh(core_axis_name="c", subcore_axis_name="s",
                                  num_cores=NC, num_subcores=NS)

    @functools.partial(
        pl.kernel, mesh=mesh,
        compiler_params=pltpu.CompilerParams(use_tc_tiling_on_sc=False),
        out_type=jax.ShapeDtypeStruct((n2, DIM), jnp.float32),
        scratch_types=[
            pltpu.VMEM((rows_w,), jnp.int32),
            pltpu.VMEM((rows_w, DIM), jnp.float32),
            pltpu.SemaphoreType.DMA,
        ],
    )
    def sc_gather(cidx_hbm, table_hbm, out_hbm, idx_v, rows_v, sem):
        wid = lax.axis_index("s") * NC + lax.axis_index("c")
        base = wid * rows_w
        pltpu.sync_copy(cidx_hbm.at[pl.ds(base, rows_w)], idx_v)
        pltpu.async_copy(table_hbm.at[idx_v], rows_v, sem).wait()
        pltpu.sync_copy(rows_v, out_hbm.at[pl.ds(base, rows_w), :])

    return sc_gather


def _assemble_block(atoms_ref, nums_ref, xyz_ref, cat_ref,
                    fm_ref, off_ref, out_ref):
    u = jnp.dot(xyz_ref[...], fm_ref[...],
                preferred_element_type=jnp.float32,
                precision=jax.lax.Precision.HIGHEST) + off_ref[...]
    out_ref[:, 0:DIM] = atoms_ref[...]
    out_ref[:, DIM:4 * DIM] = _sin_turns(u)
    out_ref[:, 4 * DIM:5 * DIM] = cat_ref[...]
    out_ref[:, 5 * DIM:6 * DIM] = nums_ref[...]


def kernel(names, x, y, z, categorical, numerical, atom_table, num_table):
    n = names.shape[0]
    table = jnp.concatenate([atom_table, num_table], axis=0)
    cidx = jnp.concatenate([names, numerical + DIM])
    gathered = _make_sc_gather(2 * n)(cidx, table)

    inv = (10000.0 ** (-2.0 * np.arange(HALF) / DIM)).astype(np.float32)
    fm = np.zeros((3, 3 * DIM), dtype=np.float32)
    off = np.zeros((1, 3 * DIM), dtype=np.float32)
    for j in range(3):
        fm[j, j * DIM:j * DIM + HALF] = inv
        fm[j, j * DIM + HALF:(j + 1) * DIM] = inv
        off[0, j * DIM + HALF:(j + 1) * DIM] = 0.25
    fm = jnp.asarray(fm)
    off = jnp.asarray(off)
    xyz = jnp.concatenate([x, y, z], axis=1)

    nb = n // BLOCK
    row_spec = lambda w: pl.BlockSpec((BLOCK, w), lambda i: (i, 0))
    fix_spec = lambda h, w: pl.BlockSpec((h, w), lambda i: (0, 0))

    return pl.pallas_call(
        _assemble_block,
        grid=(nb,),
        in_specs=[
            row_spec(DIM),
            pl.BlockSpec((BLOCK, DIM), lambda i: (nb + i, 0)),
            row_spec(3),
            row_spec(DIM),
            fix_spec(3, 3 * DIM),
            fix_spec(1, 3 * DIM),
        ],
        out_specs=row_spec(6 * DIM),
        out_shape=jax.ShapeDtypeStruct((n, 6 * DIM), jnp.float32),
        compiler_params=pltpu.CompilerParams(
            dimension_semantics=("arbitrary",)),
    )(gathered, gathered, xyz, categorical, fm, off)
